# Initial kernel scaffold; baseline (speedup 1.0000x reference)
import jax
import jax.numpy as jnp
from jax.experimental import pallas as pl


def kernel(pos, params):
    raise NotImplementedError("write your pallas kernel here")


# speedup vs baseline: 16.6523x; 16.6523x over previous
import numpy as np
from functools import partial

import jax
import jax.numpy as jnp
from jax.experimental import pallas as pl
from jax.experimental.pallas import tpu as pltpu

NUP = 8
NDOWN = 8
NELEC = NUP + NDOWN
NDIM = 3
ATOM_POS = np.array(
    [[0.0, 0.0, 0.0], [1.4, 0.0, 0.0], [0.0, 1.4, 0.0], [0.0, 0.0, 1.4]],
    dtype=np.float32,
)
NATOMS = ATOM_POS.shape[0]
NNODES_EN = NELEC + NATOMS
FEATS = 128
NRBF = 64
NLAYERS = 3
GAMMA = 10.0
RBF_CENTERS = np.linspace(0.0, 8.0, NRBF).astype(np.float32)

_spins = (np.arange(NELEC) >= NUP).astype(np.int32)
EE_ETYPE_MAT = (_spins[:, None] + _spins[None, :]).astype(np.int32).reshape(-1)
EE_NTYPES = _spins
_aq, _eq = np.divmod(np.arange(NATOMS * NELEC), NELEC)
EN_ETYPE_Q = (_spins[_eq] * NATOMS + _aq).astype(np.int32)
EN_NTYPES = np.concatenate([_spins, 2 + np.arange(NATOMS)]).astype(np.int32)

EE_MASK = (1.0 - np.eye(NELEC, dtype=np.float32)).reshape(NELEC * NELEC, 1)

BBLK = 64


def _gnn_body(pos_ref, ap_ref, cen_ref, mask_ref,
              h0e_ref, embe_ref, wrbfe_ref, we_ref, be_ref,
              w1e_ref, b1e_ref, w2e_ref, b2e_ref,
              h0n_ref, embn_ref, wrbfn_ref, wn_ref, bn_ref,
              w1n_ref, b1n_ref, w2n_ref, b2n_ref,
              out_ref):
    b = pos_ref.shape[0]
    f32 = jnp.float32

    X = pos_ref[...].T.reshape(NELEC, NDIM, b)
    ap = ap_ref[...]
    d2_ee = jnp.zeros((NELEC, NELEC, b), f32)
    d2_en = jnp.zeros((NATOMS, NELEC, b), f32)
    for k in range(NDIM):
        Xk = X[:, k, :]
        dk = Xk[:, None, :] - Xk[None, :, :]
        d2_ee = d2_ee + dk * dk
        dkn = Xk[None, :, :] - ap[:, k][:, None, None]
        d2_en = d2_en + dkn * dkn
    d_ee = jnp.sqrt(d2_ee + 1e-12).reshape(NELEC * NELEC, b)
    d_en = jnp.sqrt(d2_en + 1e-12).reshape(NATOMS * NELEC, b)

    cen = cen_ref[...][0]

    rbf_e = jnp.exp(-GAMMA * (d_ee[:, :, None] - cen[None, None, :]) ** 2)
    fe = jnp.dot(rbf_e.reshape(NELEC * NELEC * b, NRBF), wrbfe_ref[...],
                 preferred_element_type=f32)
    fe = jnp.tanh(fe.reshape(NELEC * NELEC, b, FEATS) + embe_ref[...][:, None, :])
    E4 = (fe * mask_ref[...][:, :, None]).reshape(NELEC, NELEC, b, FEATS)

    rbf_n = jnp.exp(-GAMMA * (d_en[:, :, None] - cen[None, None, :]) ** 2)
    fn = jnp.dot(rbf_n.reshape(NATOMS * NELEC * b, NRBF), wrbfn_ref[...],
                 preferred_element_type=f32)
    fn = jnp.tanh(fn.reshape(NATOMS * NELEC, b, FEATS) + embn_ref[...][:, None, :])
    EF = fn.reshape(NATOMS, NELEC, b, FEATS)

    he = jnp.broadcast_to(h0e_ref[...][:, None, :], (NELEC, b, FEATS))
    hn = jnp.broadcast_to(h0n_ref[...][:, None, :], (NNODES_EN, b, FEATS))
    for l in range(NLAYERS):
        agg_e = jnp.sum(E4 * he[None, :, :, :], axis=1)
        ze = jnp.dot(agg_e.reshape(NELEC * b, FEATS), we_ref[l],
                     preferred_element_type=f32) + be_ref[l]
        he = he + jnp.tanh(ze).reshape(NELEC, b, FEATS)

        hel = hn[:NELEC]
        hat = hn[NELEC:]
        agg_el = jnp.sum(EF * hat[:, None, :, :], axis=0)
        agg_at = jnp.sum(EF * hel[None, :, :, :], axis=1)
        aggn = jnp.concatenate([agg_el, agg_at], axis=0)
        zn = jnp.dot(aggn.reshape(NNODES_EN * b, FEATS), wn_ref[l],
                     preferred_element_type=f32) + bn_ref[l]
        hn = hn + jnp.tanh(zn).reshape(NNODES_EN, b, FEATS)

    ge = jnp.sum(he, axis=0)
    gn = jnp.sum(hn, axis=0)
    te = jnp.tanh(jnp.dot(ge, w1e_ref[...], preferred_element_type=f32) + b1e_ref[...])
    tn = jnp.tanh(jnp.dot(gn, w1n_ref[...], preferred_element_type=f32) + b1n_ref[...])
    ke = jnp.sum(te * w2e_ref[...], axis=1, keepdims=True) + b2e_ref[...]
    kn = jnp.sum(tn * w2n_ref[...], axis=1, keepdims=True) + b2n_ref[...]
    out_ref[...] = jnp.exp(ke + kn)


def _full(shape):
    nd = len(shape)
    return pl.BlockSpec(shape, lambda i, _n=nd: (0,) * _n)


@jax.jit
def kernel(pos, params):
    pe, pn = params['ee'], params['en']
    h0e = pe['node_emb'][jnp.asarray(EE_NTYPES)]
    embe = pe['edge_emb'][jnp.asarray(EE_ETYPE_MAT)]
    h0n = pn['node_emb'][jnp.asarray(EN_NTYPES)]
    embn = pn['edge_emb'][jnp.asarray(EN_ETYPE_Q)]

    weights = [
        jnp.asarray(ATOM_POS), jnp.asarray(RBF_CENTERS).reshape(1, NRBF),
        jnp.asarray(EE_MASK),
        h0e, embe, pe['w_rbf'], pe['w'], pe['b'],
        pe['w_out1'], pe['b_out1'].reshape(1, FEATS),
        pe['w_out2'].reshape(1, FEATS), pe['b_out2'].reshape(1, 1),
        h0n, embn, pn['w_rbf'], pn['w'], pn['b'],
        pn['w_out1'], pn['b_out1'].reshape(1, FEATS),
        pn['w_out2'].reshape(1, FEATS), pn['b_out2'].reshape(1, 1),
    ]

    nb = pos.shape[0]
    assert nb % BBLK == 0
    grid = (nb // BBLK,)

    return pl.pallas_call(
        _gnn_body,
        grid=grid,
        in_specs=[pl.BlockSpec((BBLK, NELEC * NDIM), lambda i: (i, 0))]
                 + [_full(w.shape) for w in weights],
        out_specs=pl.BlockSpec((BBLK, 1), lambda i: (i, 0)),
        out_shape=jax.ShapeDtypeStruct((nb, 1), jnp.float32),
        compiler_params=pltpu.CompilerParams(
            dimension_semantics=("parallel",),
        ),
    )(pos, *weights)


# speedup vs baseline: 21.1703x; 1.2713x over previous
import numpy as np
from functools import partial

import jax
import jax.numpy as jnp
from jax.experimental import pallas as pl
from jax.experimental.pallas import tpu as pltpu

NUP = 8
NDOWN = 8
NELEC = NUP + NDOWN
NDIM = 3
ATOM_POS = np.array(
    [[0.0, 0.0, 0.0], [1.4, 0.0, 0.0], [0.0, 1.4, 0.0], [0.0, 0.0, 1.4]],
    dtype=np.float32,
)
NATOMS = ATOM_POS.shape[0]
NNODES_EN = NELEC + NATOMS
FEATS = 128
NRBF = 64
NLAYERS = 3
GAMMA = 10.0
RBF_CENTERS = np.linspace(0.0, 8.0, NRBF).astype(np.float32)

_spins = (np.arange(NELEC) >= NUP).astype(np.int32)
_rows, _cols = np.triu_indices(NELEC, k=1)
NPAIRS = len(_rows)
EE_ETYPE_PAIR = (_spins[_rows] + _spins[_cols]).astype(np.int32)
EE_DIFF = np.zeros((NDIM * NPAIRS, NELEC * NDIM), np.float32)
for _k in range(NDIM):
    EE_DIFF[_k * NPAIRS + np.arange(NPAIRS), NDIM * _rows + _k] = 1.0
    EE_DIFF[_k * NPAIRS + np.arange(NPAIRS), NDIM * _cols + _k] = -1.0
EE_NTYPES = _spins
_aq, _eq = np.divmod(np.arange(NATOMS * NELEC), NELEC)
EN_ETYPE_Q = (_spins[_eq] * NATOMS + _aq).astype(np.int32)
EN_NTYPES = np.concatenate([_spins, 2 + np.arange(NATOMS)]).astype(np.int32)

BBLK = 64


def _gnn_body(pos_ref, ap_ref, cen_ref, dee_ref,
              h0e_ref, embe_ref, wrbfe_ref, we_ref, be_ref,
              w1e_ref, b1e_ref, w2e_ref, b2e_ref,
              h0n_ref, embn_ref, wrbfn_ref, wn_ref, bn_ref,
              w1n_ref, b1n_ref, w2n_ref, b2n_ref,
              out_ref, e4_ref):
    b = pos_ref.shape[0]
    f32 = jnp.float32

    Xt = pos_ref[...].T
    X = Xt.reshape(NELEC, NDIM, b)
    ap = ap_ref[...]
    dif = jnp.dot(dee_ref[...], Xt,
                  preferred_element_type=f32).reshape(NDIM, NPAIRS, b)
    d_ee = jnp.sqrt(jnp.sum(dif * dif, axis=0) + 1e-12)
    d2_en = jnp.zeros((NATOMS, NELEC, b), f32)
    for k in range(NDIM):
        Xk = X[:, k, :]
        dkn = Xk[None, :, :] - ap[:, k][:, None, None]
        d2_en = d2_en + dkn * dkn
    d_en = jnp.sqrt(d2_en + 1e-12).reshape(NATOMS * NELEC, b)

    cen = cen_ref[...][0]

    rbf_e = jnp.exp(-GAMMA * (d_ee[:, :, None] - cen[None, None, :]) ** 2)
    fe = jnp.dot(rbf_e.reshape(NPAIRS * b, NRBF), wrbfe_ref[...],
                 preferred_element_type=f32)
    fe = jnp.tanh(fe.reshape(NPAIRS, b, FEATS) + embe_ref[...][:, None, :])

    rbf_n = jnp.exp(-GAMMA * (d_en[:, :, None] - cen[None, None, :]) ** 2)
    fn = jnp.dot(rbf_n.reshape(NATOMS * NELEC * b, NRBF), wrbfn_ref[...],
                 preferred_element_type=f32)
    fn = jnp.tanh(fn.reshape(NATOMS * NELEC, b, FEATS) + embn_ref[...][:, None, :])
    EF = fn.reshape(NATOMS, NELEC, b, FEATS)

    zrow = jnp.zeros((b, FEATS), f32)
    for i in range(NELEC):
        e4_ref[i * NELEC + i] = zrow
    for p in range(NPAIRS):
        r = int(_rows[p]); c = int(_cols[p])
        row = fe[p]
        e4_ref[r * NELEC + c] = row
        e4_ref[c * NELEC + r] = row
    E4 = e4_ref[...].reshape(NELEC, NELEC, b, FEATS)

    he = jnp.broadcast_to(h0e_ref[...][:, None, :], (NELEC, b, FEATS))
    hn = jnp.broadcast_to(h0n_ref[...][:, None, :], (NNODES_EN, b, FEATS))
    for l in range(NLAYERS):
        agg_e = jnp.sum(E4 * he[None, :, :, :], axis=1)
        ze = jnp.dot(agg_e.reshape(NELEC * b, FEATS), we_ref[l],
                     preferred_element_type=f32) + be_ref[l]
        he = he + jnp.tanh(ze).reshape(NELEC, b, FEATS)

        hel = hn[:NELEC]
        hat = hn[NELEC:]
        agg_el = jnp.sum(EF * hat[:, None, :, :], axis=0)
        agg_at = jnp.sum(EF * hel[None, :, :, :], axis=1)
        aggn = jnp.concatenate([agg_el, agg_at], axis=0)
        zn = jnp.dot(aggn.reshape(NNODES_EN * b, FEATS), wn_ref[l],
                     preferred_element_type=f32) + bn_ref[l]
        hn = hn + jnp.tanh(zn).reshape(NNODES_EN, b, FEATS)

    ge = jnp.sum(he, axis=0)
    gn = jnp.sum(hn, axis=0)
    te = jnp.tanh(jnp.dot(ge, w1e_ref[...], preferred_element_type=f32) + b1e_ref[...])
    tn = jnp.tanh(jnp.dot(gn, w1n_ref[...], preferred_element_type=f32) + b1n_ref[...])
    ke = jnp.sum(te * w2e_ref[...], axis=1, keepdims=True) + b2e_ref[...]
    kn = jnp.sum(tn * w2n_ref[...], axis=1, keepdims=True) + b2n_ref[...]
    out_ref[...] = jnp.exp(ke + kn)


def _full(shape):
    nd = len(shape)
    return pl.BlockSpec(shape, lambda i, _n=nd: (0,) * _n)


@jax.jit
def kernel(pos, params):
    pe, pn = params['ee'], params['en']
    h0e = pe['node_emb'][jnp.asarray(EE_NTYPES)]
    embe = pe['edge_emb'][jnp.asarray(EE_ETYPE_PAIR)]
    h0n = pn['node_emb'][jnp.asarray(EN_NTYPES)]
    embn = pn['edge_emb'][jnp.asarray(EN_ETYPE_Q)]

    weights = [
        jnp.asarray(ATOM_POS), jnp.asarray(RBF_CENTERS).reshape(1, NRBF),
        jnp.asarray(EE_DIFF),
        h0e, embe, pe['w_rbf'], pe['w'], pe['b'],
        pe['w_out1'], pe['b_out1'].reshape(1, FEATS),
        pe['w_out2'].reshape(1, FEATS), pe['b_out2'].reshape(1, 1),
        h0n, embn, pn['w_rbf'], pn['w'], pn['b'],
        pn['w_out1'], pn['b_out1'].reshape(1, FEATS),
        pn['w_out2'].reshape(1, FEATS), pn['b_out2'].reshape(1, 1),
    ]

    nb = pos.shape[0]
    assert nb % BBLK == 0
    grid = (nb // BBLK,)

    return pl.pallas_call(
        _gnn_body,
        grid=grid,
        in_specs=[pl.BlockSpec((BBLK, NELEC * NDIM), lambda i: (i, 0))]
                 + [_full(w.shape) for w in weights],
        out_specs=pl.BlockSpec((BBLK, 1), lambda i: (i, 0)),
        out_shape=jax.ShapeDtypeStruct((nb, 1), jnp.float32),
        scratch_shapes=[pltpu.VMEM((NELEC * NELEC, BBLK, FEATS), jnp.float32)],
        compiler_params=pltpu.CompilerParams(
            dimension_semantics=("parallel",),
        ),
    )(pos, *weights)


# speedup vs baseline: 21.5235x; 1.0167x over previous
import numpy as np
from functools import partial

import jax
import jax.numpy as jnp
from jax.experimental import pallas as pl
from jax.experimental.pallas import tpu as pltpu

NUP = 8
NDOWN = 8
NELEC = NUP + NDOWN
NDIM = 3
ATOM_POS = np.array(
    [[0.0, 0.0, 0.0], [1.4, 0.0, 0.0], [0.0, 1.4, 0.0], [0.0, 0.0, 1.4]],
    dtype=np.float32,
)
NATOMS = ATOM_POS.shape[0]
NNODES_EN = NELEC + NATOMS
FEATS = 128
NRBF = 64
NLAYERS = 3
GAMMA = 10.0
RBF_CENTERS = np.linspace(0.0, 8.0, NRBF).astype(np.float32)

_spins = (np.arange(NELEC) >= NUP).astype(np.int32)
_rows, _cols = np.triu_indices(NELEC, k=1)
NPAIRS = len(_rows)
EE_ETYPE_PAIR = (_spins[_rows] + _spins[_cols]).astype(np.int32)
EE_DIFF = np.zeros((NDIM * NPAIRS, NELEC * NDIM), np.float32)
for _k in range(NDIM):
    EE_DIFF[_k * NPAIRS + np.arange(NPAIRS), NDIM * _rows + _k] = 1.0
    EE_DIFF[_k * NPAIRS + np.arange(NPAIRS), NDIM * _cols + _k] = -1.0
EE_NTYPES = _spins
_aq, _eq = np.divmod(np.arange(NATOMS * NELEC), NELEC)
EN_ETYPE_Q = (_spins[_eq] * NATOMS + _aq).astype(np.int32)
EN_NTYPES = np.concatenate([_spins, 2 + np.arange(NATOMS)]).astype(np.int32)

BBLK = 64


def _gnn_body(pos_ref, ap_ref, cen_ref, dee_ref,
              h0e_ref, embe_ref, wrbfe_ref, we_ref, be_ref,
              w1e_ref, b1e_ref, w2e_ref, b2e_ref,
              h0n_ref, embn_ref, wrbfn_ref, wn_ref, bn_ref,
              w1n_ref, b1n_ref, w2n_ref, b2n_ref,
              out_ref, e4_ref):
    b = pos_ref.shape[0]
    f32 = jnp.float32

    Xt = pos_ref[...].T
    X = Xt.reshape(NELEC, NDIM, b)
    ap = ap_ref[...]
    dif = jnp.dot(dee_ref[...], Xt,
                  preferred_element_type=f32).reshape(NDIM, NPAIRS, b)
    G = GAMMA
    d_ee = jnp.sqrt(G * jnp.sum(dif * dif, axis=0) + G * 1e-12)
    d2_en = jnp.zeros((NATOMS, NELEC, b), f32)
    for k in range(NDIM):
        Xk = X[:, k, :]
        dkn = Xk[None, :, :] - ap[:, k][:, None, None]
        d2_en = d2_en + dkn * dkn
    d_en = jnp.sqrt(G * d2_en + G * 1e-12).reshape(NATOMS * NELEC, b)

    cen = cen_ref[...][0]

    t_e = d_ee[:, :, None] - cen[None, None, :]
    rbf_e = jnp.exp(-(t_e * t_e))
    fe = jnp.dot(rbf_e.reshape(NPAIRS * b, NRBF), wrbfe_ref[...],
                 preferred_element_type=f32)
    fe = jnp.tanh(fe.reshape(NPAIRS, b, FEATS) + embe_ref[...][:, None, :])

    t_n = d_en[:, :, None] - cen[None, None, :]
    rbf_n = jnp.exp(-(t_n * t_n))
    fn = jnp.dot(rbf_n.reshape(NATOMS * NELEC * b, NRBF), wrbfn_ref[...],
                 preferred_element_type=f32)
    fn = jnp.tanh(fn.reshape(NATOMS * NELEC, b, FEATS) + embn_ref[...][:, None, :])
    EF = fn.reshape(NATOMS, NELEC, b, FEATS)

    zrow = jnp.zeros((b, FEATS), f32)
    for i in range(NELEC):
        e4_ref[i * NELEC + i] = zrow
    for p in range(NPAIRS):
        r = int(_rows[p]); c = int(_cols[p])
        row = fe[p]
        e4_ref[r * NELEC + c] = row
        e4_ref[c * NELEC + r] = row
    E4 = e4_ref[...].reshape(NELEC, NELEC, b, FEATS)

    he = jnp.broadcast_to(h0e_ref[...][:, None, :], (NELEC, b, FEATS))
    hn = jnp.broadcast_to(h0n_ref[...][:, None, :], (NNODES_EN, b, FEATS))
    for l in range(NLAYERS):
        agg_e = E4[:, 0] * he[0]
        for s in range(1, NELEC):
            agg_e = agg_e + E4[:, s] * he[s]
        ze = jnp.dot(agg_e.reshape(NELEC * b, FEATS), we_ref[l],
                     preferred_element_type=f32) + be_ref[l]
        he = he + jnp.tanh(ze).reshape(NELEC, b, FEATS)

        hel = hn[:NELEC]
        hat = hn[NELEC:]
        agg_el = EF[0] * hat[0]
        agg_at = EF[:, 0] * hel[0]
        for a in range(1, NATOMS):
            agg_el = agg_el + EF[a] * hat[a]
        for s in range(1, NELEC):
            agg_at = agg_at + EF[:, s] * hel[s]
        aggn = jnp.concatenate([agg_el, agg_at], axis=0)
        zn = jnp.dot(aggn.reshape(NNODES_EN * b, FEATS), wn_ref[l],
                     preferred_element_type=f32) + bn_ref[l]
        hn = hn + jnp.tanh(zn).reshape(NNODES_EN, b, FEATS)

    ge = jnp.sum(he, axis=0)
    gn = jnp.sum(hn, axis=0)
    te = jnp.tanh(jnp.dot(ge, w1e_ref[...], preferred_element_type=f32) + b1e_ref[...])
    tn = jnp.tanh(jnp.dot(gn, w1n_ref[...], preferred_element_type=f32) + b1n_ref[...])
    ke = jnp.sum(te * w2e_ref[...], axis=1, keepdims=True) + b2e_ref[...]
    kn = jnp.sum(tn * w2n_ref[...], axis=1, keepdims=True) + b2n_ref[...]
    out_ref[...] = jnp.exp(ke + kn)


def _full(shape):
    nd = len(shape)
    return pl.BlockSpec(shape, lambda i, _n=nd: (0,) * _n)


@jax.jit
def kernel(pos, params):
    pe, pn = params['ee'], params['en']
    h0e = pe['node_emb'][jnp.asarray(EE_NTYPES)]
    embe = pe['edge_emb'][jnp.asarray(EE_ETYPE_PAIR)]
    h0n = pn['node_emb'][jnp.asarray(EN_NTYPES)]
    embn = pn['edge_emb'][jnp.asarray(EN_ETYPE_Q)]

    weights = [
        jnp.asarray(ATOM_POS),
        jnp.asarray(np.sqrt(GAMMA) * RBF_CENTERS).reshape(1, NRBF),
        jnp.asarray(EE_DIFF),
        h0e, embe, pe['w_rbf'], pe['w'], pe['b'],
        pe['w_out1'], pe['b_out1'].reshape(1, FEATS),
        pe['w_out2'].reshape(1, FEATS), pe['b_out2'].reshape(1, 1),
        h0n, embn, pn['w_rbf'], pn['w'], pn['b'],
        pn['w_out1'], pn['b_out1'].reshape(1, FEATS),
        pn['w_out2'].reshape(1, FEATS), pn['b_out2'].reshape(1, 1),
    ]

    nb = pos.shape[0]
    assert nb % BBLK == 0
    grid = (nb // BBLK,)

    return pl.pallas_call(
        _gnn_body,
        grid=grid,
        in_specs=[pl.BlockSpec((BBLK, NELEC * NDIM), lambda i: (i, 0))]
                 + [_full(w.shape) for w in weights],
        out_specs=pl.BlockSpec((BBLK, 1), lambda i: (i, 0)),
        out_shape=jax.ShapeDtypeStruct((nb, 1), jnp.float32),
        scratch_shapes=[pltpu.VMEM((NELEC * NELEC, BBLK, FEATS), jnp.float32)],
        compiler_params=pltpu.CompilerParams(
            dimension_semantics=("parallel",),
        ),
    )(pos, *weights)


# speedup vs baseline: 23.2944x; 1.0823x over previous
import numpy as np
from functools import partial

import jax
import jax.numpy as jnp
from jax.experimental import pallas as pl
from jax.experimental.pallas import tpu as pltpu

NUP = 8
NDOWN = 8
NELEC = NUP + NDOWN
NDIM = 3
ATOM_POS = np.array(
    [[0.0, 0.0, 0.0], [1.4, 0.0, 0.0], [0.0, 1.4, 0.0], [0.0, 0.0, 1.4]],
    dtype=np.float32,
)
NATOMS = ATOM_POS.shape[0]
NNODES_EN = NELEC + NATOMS
FEATS = 128
NRBF = 64
NLAYERS = 3
GAMMA = 10.0
RBF_CENTERS = np.linspace(0.0, 8.0, NRBF).astype(np.float32)

_spins = (np.arange(NELEC) >= NUP).astype(np.int32)
_rows, _cols = np.triu_indices(NELEC, k=1)
NPAIRS = len(_rows)
EE_ETYPE_PAIR = (_spins[_rows] + _spins[_cols]).astype(np.int32)
EE_DIFF = np.zeros((NDIM * NPAIRS, NELEC * NDIM), np.float32)
for _k in range(NDIM):
    EE_DIFF[_k * NPAIRS + np.arange(NPAIRS), NDIM * _rows + _k] = 1.0
    EE_DIFF[_k * NPAIRS + np.arange(NPAIRS), NDIM * _cols + _k] = -1.0
EE_NTYPES = _spins
_aq, _eq = np.divmod(np.arange(NATOMS * NELEC), NELEC)
EN_ETYPE_Q = (_spins[_eq] * NATOMS + _aq).astype(np.int32)
EN_NTYPES = np.concatenate([_spins, 2 + np.arange(NATOMS)]).astype(np.int32)

BBLK = 128


def _gnn_body(pos_ref, ap_ref, cen_ref, dee_ref,
              h0e_ref, embe_ref, wrbfe_ref, we_ref, be_ref,
              w1e_ref, b1e_ref, w2e_ref, b2e_ref,
              h0n_ref, embn_ref, wrbfn_ref, wn_ref, bn_ref,
              w1n_ref, b1n_ref, w2n_ref, b2n_ref,
              out_ref, e4_ref):
    b = pos_ref.shape[0]
    f32 = jnp.float32

    Xt = pos_ref[...].T
    X = Xt.reshape(NELEC, NDIM, b)
    ap = ap_ref[...]
    dif = jnp.dot(dee_ref[...], Xt,
                  preferred_element_type=f32).reshape(NDIM, NPAIRS, b)
    G = GAMMA
    d_ee = jnp.sqrt(G * jnp.sum(dif * dif, axis=0) + G * 1e-12)
    d2_en = jnp.zeros((NATOMS, NELEC, b), f32)
    for k in range(NDIM):
        Xk = X[:, k, :]
        dkn = Xk[None, :, :] - ap[:, k][:, None, None]
        d2_en = d2_en + dkn * dkn
    d_en = jnp.sqrt(G * d2_en + G * 1e-12).reshape(NATOMS * NELEC, b)

    cen = cen_ref[...][0]

    t_e = d_ee[:, :, None] - cen[None, None, :]
    rbf_e = jnp.exp(-(t_e * t_e))
    fe = jnp.dot(rbf_e.reshape(NPAIRS * b, NRBF), wrbfe_ref[...],
                 preferred_element_type=f32)
    fe = jnp.tanh(fe.reshape(NPAIRS, b, FEATS) + embe_ref[...][:, None, :])

    t_n = d_en[:, :, None] - cen[None, None, :]
    rbf_n = jnp.exp(-(t_n * t_n))
    fn = jnp.dot(rbf_n.reshape(NATOMS * NELEC * b, NRBF), wrbfn_ref[...],
                 preferred_element_type=f32)
    fn = jnp.tanh(fn.reshape(NATOMS * NELEC, b, FEATS) + embn_ref[...][:, None, :])
    EF = fn.reshape(NATOMS, NELEC, b, FEATS)

    zrow = jnp.zeros((b, FEATS), f32)
    for i in range(NELEC):
        e4_ref[i * NELEC + i] = zrow
    for p in range(NPAIRS):
        r = int(_rows[p]); c = int(_cols[p])
        row = fe[p]
        e4_ref[r * NELEC + c] = row
        e4_ref[c * NELEC + r] = row
    E4 = e4_ref[...].reshape(NELEC, NELEC, b, FEATS)

    he = jnp.broadcast_to(h0e_ref[...][:, None, :], (NELEC, b, FEATS))
    hn = jnp.broadcast_to(h0n_ref[...][:, None, :], (NNODES_EN, b, FEATS))
    for l in range(NLAYERS):
        agg_e = E4[:, 0] * he[0]
        for s in range(1, NELEC):
            agg_e = agg_e + E4[:, s] * he[s]
        ze = jnp.dot(agg_e.reshape(NELEC * b, FEATS), we_ref[l],
                     preferred_element_type=f32) + be_ref[l]
        he = he + jnp.tanh(ze).reshape(NELEC, b, FEATS)

        hel = hn[:NELEC]
        hat = hn[NELEC:]
        agg_el = EF[0] * hat[0]
        agg_at = EF[:, 0] * hel[0]
        for a in range(1, NATOMS):
            agg_el = agg_el + EF[a] * hat[a]
        for s in range(1, NELEC):
            agg_at = agg_at + EF[:, s] * hel[s]
        aggn = jnp.concatenate([agg_el, agg_at], axis=0)
        zn = jnp.dot(aggn.reshape(NNODES_EN * b, FEATS), wn_ref[l],
                     preferred_element_type=f32) + bn_ref[l]
        hn = hn + jnp.tanh(zn).reshape(NNODES_EN, b, FEATS)

    ge = jnp.sum(he, axis=0)
    gn = jnp.sum(hn, axis=0)
    te = jnp.tanh(jnp.dot(ge, w1e_ref[...], preferred_element_type=f32) + b1e_ref[...])
    tn = jnp.tanh(jnp.dot(gn, w1n_ref[...], preferred_element_type=f32) + b1n_ref[...])
    ke = jnp.sum(te * w2e_ref[...], axis=1, keepdims=True) + b2e_ref[...]
    kn = jnp.sum(tn * w2n_ref[...], axis=1, keepdims=True) + b2n_ref[...]
    out_ref[...] = jnp.exp(ke + kn)


def _full(shape):
    nd = len(shape)
    return pl.BlockSpec(shape, lambda i, _n=nd: (0,) * _n)


@jax.jit
def kernel(pos, params):
    pe, pn = params['ee'], params['en']
    h0e = pe['node_emb'][jnp.asarray(EE_NTYPES)]
    embe = pe['edge_emb'][jnp.asarray(EE_ETYPE_PAIR)]
    h0n = pn['node_emb'][jnp.asarray(EN_NTYPES)]
    embn = pn['edge_emb'][jnp.asarray(EN_ETYPE_Q)]

    weights = [
        jnp.asarray(ATOM_POS),
        jnp.asarray(np.sqrt(GAMMA) * RBF_CENTERS).reshape(1, NRBF),
        jnp.asarray(EE_DIFF),
        h0e, embe, pe['w_rbf'], pe['w'], pe['b'],
        pe['w_out1'], pe['b_out1'].reshape(1, FEATS),
        pe['w_out2'].reshape(1, FEATS), pe['b_out2'].reshape(1, 1),
        h0n, embn, pn['w_rbf'], pn['w'], pn['b'],
        pn['w_out1'], pn['b_out1'].reshape(1, FEATS),
        pn['w_out2'].reshape(1, FEATS), pn['b_out2'].reshape(1, 1),
    ]

    nb = pos.shape[0]
    assert nb % BBLK == 0
    grid = (nb // BBLK,)

    return pl.pallas_call(
        _gnn_body,
        grid=grid,
        in_specs=[pl.BlockSpec((BBLK, NELEC * NDIM), lambda i: (i, 0))]
                 + [_full(w.shape) for w in weights],
        out_specs=pl.BlockSpec((BBLK, 1), lambda i: (i, 0)),
        out_shape=jax.ShapeDtypeStruct((nb, 1), jnp.float32),
        scratch_shapes=[pltpu.VMEM((NELEC * NELEC, BBLK, FEATS), jnp.float32)],
        compiler_params=pltpu.CompilerParams(
            dimension_semantics=("parallel",),
        ),
    )(pos, *weights)


# speedup vs baseline: 23.3407x; 1.0020x over previous
import numpy as np
from functools import partial

import jax
import jax.numpy as jnp
from jax.experimental import pallas as pl
from jax.experimental.pallas import tpu as pltpu

NUP = 8
NDOWN = 8
NELEC = NUP + NDOWN
NDIM = 3
ATOM_POS = np.array(
    [[0.0, 0.0, 0.0], [1.4, 0.0, 0.0], [0.0, 1.4, 0.0], [0.0, 0.0, 1.4]],
    dtype=np.float32,
)
NATOMS = ATOM_POS.shape[0]
NNODES_EN = NELEC + NATOMS
FEATS = 128
NRBF = 64
NLAYERS = 3
GAMMA = 10.0
RBF_CENTERS = np.linspace(0.0, 8.0, NRBF).astype(np.float32)

_spins = (np.arange(NELEC) >= NUP).astype(np.int32)
_rows, _cols = np.triu_indices(NELEC, k=1)
NPAIRS = len(_rows)
EE_ETYPE_PAIR = (_spins[_rows] + _spins[_cols]).astype(np.int32)
EE_DIFF = np.zeros((NDIM * NPAIRS, NELEC * NDIM), np.float32)
for _k in range(NDIM):
    EE_DIFF[_k * NPAIRS + np.arange(NPAIRS), NDIM * _rows + _k] = 1.0
    EE_DIFF[_k * NPAIRS + np.arange(NPAIRS), NDIM * _cols + _k] = -1.0
EE_NTYPES = _spins
_aq, _eq = np.divmod(np.arange(NATOMS * NELEC), NELEC)
EN_ETYPE_Q = (_spins[_eq] * NATOMS + _aq).astype(np.int32)
EN_NTYPES = np.concatenate([_spins, 2 + np.arange(NATOMS)]).astype(np.int32)

BBLK = 128


def _gnn_body(pos_ref, ap_ref, cen_ref, dee_ref,
              h0e_ref, embe_ref, wrbfe_ref, we_ref, be_ref,
              w1e_ref, b1e_ref, w2e_ref, b2e_ref,
              h0n_ref, embn_ref, wrbfn_ref, wn_ref, bn_ref,
              w1n_ref, b1n_ref, w2n_ref, b2n_ref,
              out_ref, e4_ref):
    b = pos_ref.shape[0]
    f32 = jnp.float32

    Xt = pos_ref[...].T
    X = Xt.reshape(NELEC, NDIM, b)
    ap = ap_ref[...]
    dif = jnp.dot(dee_ref[...], Xt,
                  preferred_element_type=f32).reshape(NDIM, NPAIRS, b)
    G = GAMMA
    d_ee = jnp.sqrt(G * jnp.sum(dif * dif, axis=0) + G * 1e-12)
    d2_en = jnp.zeros((NATOMS, NELEC, b), f32)
    for k in range(NDIM):
        Xk = X[:, k, :]
        dkn = Xk[None, :, :] - ap[:, k][:, None, None]
        d2_en = d2_en + dkn * dkn
    d_en = jnp.sqrt(G * d2_en + G * 1e-12).reshape(NATOMS * NELEC, b)

    cen = cen_ref[...][0]

    t_e = d_ee[:, :, None] - cen[None, None, :]
    rbf_e = jnp.exp(-(t_e * t_e))
    fe = jnp.dot(rbf_e.reshape(NPAIRS * b, NRBF), wrbfe_ref[...],
                 preferred_element_type=f32)
    fe = jnp.tanh(fe.reshape(NPAIRS, b, FEATS) + embe_ref[...][:, None, :])

    t_n = d_en[:, :, None] - cen[None, None, :]
    rbf_n = jnp.exp(-(t_n * t_n))
    fn = jnp.dot(rbf_n.reshape(NATOMS * NELEC * b, NRBF), wrbfn_ref[...],
                 preferred_element_type=f32)
    fn = jnp.tanh(fn.reshape(NATOMS * NELEC, b, FEATS) + embn_ref[...][:, None, :])
    EF = fn.reshape(NATOMS, NELEC, b, FEATS)

    zrow = jnp.zeros((b, FEATS), f32)
    for i in range(NELEC):
        e4_ref[i * NELEC + i] = zrow
    for p in range(NPAIRS):
        r = int(_rows[p]); c = int(_cols[p])
        row = fe[p]
        e4_ref[r * NELEC + c] = row
        e4_ref[c * NELEC + r] = row

    DB = 4
    he = jnp.broadcast_to(h0e_ref[...][:, None, :], (NELEC, b, FEATS))
    hn = jnp.broadcast_to(h0n_ref[...][:, None, :], (NNODES_EN, b, FEATS))
    for l in range(NLAYERS):
        agg_rows = []
        for d0 in range(0, NELEC, DB):
            accs = [None] * DB
            for s in range(NELEC):
                hes = he[s]
                for j in range(DB):
                    t = e4_ref[(d0 + j) * NELEC + s] * hes
                    accs[j] = t if accs[j] is None else accs[j] + t
            agg_rows += accs
        agg_e = jnp.stack(agg_rows)
        ze = jnp.dot(agg_e.reshape(NELEC * b, FEATS), we_ref[l],
                     preferred_element_type=f32) + be_ref[l]
        he = he + jnp.tanh(ze).reshape(NELEC, b, FEATS)

        hel = hn[:NELEC]
        hat = hn[NELEC:]
        el_rows = []
        for e0 in range(0, NELEC, DB):
            accs = [None] * DB
            for a in range(NATOMS):
                hata = hat[a]
                for j in range(DB):
                    t = EF[a, e0 + j] * hata
                    accs[j] = t if accs[j] is None else accs[j] + t
            el_rows += accs
        at_accs = [None] * NATOMS
        for e in range(NELEC):
            hele = hel[e]
            for a in range(NATOMS):
                t = EF[a, e] * hele
                at_accs[a] = t if at_accs[a] is None else at_accs[a] + t
        aggn = jnp.stack(el_rows + at_accs)
        zn = jnp.dot(aggn.reshape(NNODES_EN * b, FEATS), wn_ref[l],
                     preferred_element_type=f32) + bn_ref[l]
        hn = hn + jnp.tanh(zn).reshape(NNODES_EN, b, FEATS)

    ge = jnp.sum(he, axis=0)
    gn = jnp.sum(hn, axis=0)
    te = jnp.tanh(jnp.dot(ge, w1e_ref[...], preferred_element_type=f32) + b1e_ref[...])
    tn = jnp.tanh(jnp.dot(gn, w1n_ref[...], preferred_element_type=f32) + b1n_ref[...])
    ke = jnp.sum(te * w2e_ref[...], axis=1, keepdims=True) + b2e_ref[...]
    kn = jnp.sum(tn * w2n_ref[...], axis=1, keepdims=True) + b2n_ref[...]
    out_ref[...] = jnp.exp(ke + kn)


def _full(shape):
    nd = len(shape)
    return pl.BlockSpec(shape, lambda i, _n=nd: (0,) * _n)


@jax.jit
def kernel(pos, params):
    pe, pn = params['ee'], params['en']
    h0e = pe['node_emb'][jnp.asarray(EE_NTYPES)]
    embe = pe['edge_emb'][jnp.asarray(EE_ETYPE_PAIR)]
    h0n = pn['node_emb'][jnp.asarray(EN_NTYPES)]
    embn = pn['edge_emb'][jnp.asarray(EN_ETYPE_Q)]

    weights = [
        jnp.asarray(ATOM_POS),
        jnp.asarray(np.sqrt(GAMMA) * RBF_CENTERS).reshape(1, NRBF),
        jnp.asarray(EE_DIFF),
        h0e, embe, pe['w_rbf'], pe['w'], pe['b'],
        pe['w_out1'], pe['b_out1'].reshape(1, FEATS),
        pe['w_out2'].reshape(1, FEATS), pe['b_out2'].reshape(1, 1),
        h0n, embn, pn['w_rbf'], pn['w'], pn['b'],
        pn['w_out1'], pn['b_out1'].reshape(1, FEATS),
        pn['w_out2'].reshape(1, FEATS), pn['b_out2'].reshape(1, 1),
    ]

    nb = pos.shape[0]
    assert nb % BBLK == 0
    grid = (nb // BBLK,)

    return pl.pallas_call(
        _gnn_body,
        grid=grid,
        in_specs=[pl.BlockSpec((BBLK, NELEC * NDIM), lambda i: (i, 0))]
                 + [_full(w.shape) for w in weights],
        out_specs=pl.BlockSpec((BBLK, 1), lambda i: (i, 0)),
        out_shape=jax.ShapeDtypeStruct((nb, 1), jnp.float32),
        scratch_shapes=[pltpu.VMEM((NELEC * NELEC, BBLK, FEATS), jnp.float32)],
        compiler_params=pltpu.CompilerParams(
            dimension_semantics=("parallel",),
        ),
    )(pos, *weights)


# speedup vs baseline: 24.4185x; 1.0462x over previous
import numpy as np
from functools import partial

import jax
import jax.numpy as jnp
from jax.experimental import pallas as pl
from jax.experimental.pallas import tpu as pltpu

NUP = 8
NDOWN = 8
NELEC = NUP + NDOWN
NDIM = 3
ATOM_POS = np.array(
    [[0.0, 0.0, 0.0], [1.4, 0.0, 0.0], [0.0, 1.4, 0.0], [0.0, 0.0, 1.4]],
    dtype=np.float32,
)
NATOMS = ATOM_POS.shape[0]
NNODES_EN = NELEC + NATOMS
FEATS = 128
NRBF = 64
NLAYERS = 3
GAMMA = 10.0
RBF_CENTERS = np.linspace(0.0, 8.0, NRBF).astype(np.float32)

_spins = (np.arange(NELEC) >= NUP).astype(np.int32)
_rows, _cols = np.triu_indices(NELEC, k=1)
NPAIRS = len(_rows)
EE_ETYPE_PAIR = (_spins[_rows] + _spins[_cols]).astype(np.int32)
EE_DIFF = np.zeros((NDIM * NPAIRS, NELEC * NDIM), np.float32)
for _k in range(NDIM):
    EE_DIFF[_k * NPAIRS + np.arange(NPAIRS), NDIM * _rows + _k] = 1.0
    EE_DIFF[_k * NPAIRS + np.arange(NPAIRS), NDIM * _cols + _k] = -1.0
EE_NTYPES = _spins
_aq, _eq = np.divmod(np.arange(NATOMS * NELEC), NELEC)
EN_ETYPE_Q = (_spins[_eq] * NATOMS + _aq).astype(np.int32)
EN_NTYPES = np.concatenate([_spins, 2 + np.arange(NATOMS)]).astype(np.int32)

BBLK = 128


def _gnn_body(pos_ref, ap_ref, cen_ref, dee_ref,
              h0e_ref, embe_ref, wrbfe_ref, we_ref, be_ref,
              w1e_ref, b1e_ref, w2e_ref, b2e_ref,
              h0n_ref, embn_ref, wrbfn_ref, wn_ref, bn_ref,
              w1n_ref, b1n_ref, w2n_ref, b2n_ref,
              out_ref, e4_ref):
    b = pos_ref.shape[0]
    f32 = jnp.float32

    Xt = pos_ref[...].T
    X = Xt.reshape(NELEC, NDIM, b)
    ap = ap_ref[...]
    dif = jnp.dot(dee_ref[...], Xt,
                  preferred_element_type=f32).reshape(NDIM, NPAIRS, b)
    G = GAMMA
    d_ee = jnp.sqrt(G * jnp.sum(dif * dif, axis=0) + G * 1e-12)
    d2_en = jnp.zeros((NATOMS, NELEC, b), f32)
    for k in range(NDIM):
        Xk = X[:, k, :]
        dkn = Xk[None, :, :] - ap[:, k][:, None, None]
        d2_en = d2_en + dkn * dkn
    d_en = jnp.sqrt(G * d2_en + G * 1e-12).reshape(NATOMS * NELEC, b)

    cen = cen_ref[...][0]

    t_e = d_ee[:, :, None] - cen[None, None, :]
    rbf_e = jnp.exp(-(t_e * t_e))
    fe = jnp.dot(rbf_e.reshape(NPAIRS * b, NRBF), wrbfe_ref[...],
                 preferred_element_type=f32)
    fe = jnp.tanh(fe.reshape(NPAIRS, b, FEATS) + embe_ref[...][:, None, :])

    t_n = d_en[:, :, None] - cen[None, None, :]
    rbf_n = jnp.exp(-(t_n * t_n))
    fn = jnp.dot(rbf_n.reshape(NATOMS * NELEC * b, NRBF), wrbfn_ref[...],
                 preferred_element_type=f32)
    fn = jnp.tanh(fn.reshape(NATOMS * NELEC, b, FEATS) + embn_ref[...][:, None, :])
    EF = fn.reshape(NATOMS, NELEC, b, FEATS)

    zrow = jnp.zeros((b, FEATS), f32)
    for i in range(NELEC):
        e4_ref[i * NELEC + i] = zrow
    for p in range(NPAIRS):
        r = int(_rows[p]); c = int(_cols[p])
        row = fe[p]
        e4_ref[r * NELEC + c] = row
        e4_ref[c * NELEC + r] = row

    DB = 4
    he = jnp.broadcast_to(h0e_ref[...][:, None, :], (NELEC, b, FEATS))
    hn = jnp.broadcast_to(h0n_ref[...][:, None, :], (NNODES_EN, b, FEATS))
    for l in range(NLAYERS):
        if l == 0:
            hu = he[0]
            hv = he[NUP]
            agg_rows = []
            for d in range(NELEC):
                su = e4_ref[d * NELEC + 0]
                for s in range(1, NUP):
                    su = su + e4_ref[d * NELEC + s]
                sv = e4_ref[d * NELEC + NUP]
                for s in range(NUP + 1, NELEC):
                    sv = sv + e4_ref[d * NELEC + s]
                agg_rows.append(su * hu + sv * hv)
        else:
            agg_rows = []
            for d0 in range(0, NELEC, DB):
                accs = [None] * DB
                for s in range(NELEC):
                    hes = he[s]
                    for j in range(DB):
                        t = e4_ref[(d0 + j) * NELEC + s] * hes
                        accs[j] = t if accs[j] is None else accs[j] + t
                agg_rows += accs
        agg_e = jnp.stack(agg_rows)
        ze = jnp.dot(agg_e.reshape(NELEC * b, FEATS), we_ref[l],
                     preferred_element_type=f32) + be_ref[l]
        he = he + jnp.tanh(ze).reshape(NELEC, b, FEATS)

        hel = hn[:NELEC]
        hat = hn[NELEC:]
        el_rows = []
        for e0 in range(0, NELEC, DB):
            accs = [None] * DB
            for a in range(NATOMS):
                hata = hat[a]
                for j in range(DB):
                    t = EF[a, e0 + j] * hata
                    accs[j] = t if accs[j] is None else accs[j] + t
            el_rows += accs
        if l == 0:
            hnu = hel[0]
            hnv = hel[NUP]
            at_accs = []
            for a in range(NATOMS):
                su = EF[a, 0]
                for e in range(1, NUP):
                    su = su + EF[a, e]
                sv = EF[a, NUP]
                for e in range(NUP + 1, NELEC):
                    sv = sv + EF[a, e]
                at_accs.append(su * hnu + sv * hnv)
        else:
            at_accs = [None] * NATOMS
            for e in range(NELEC):
                hele = hel[e]
                for a in range(NATOMS):
                    t = EF[a, e] * hele
                    at_accs[a] = t if at_accs[a] is None else at_accs[a] + t
        aggn = jnp.stack(el_rows + at_accs)
        zn = jnp.dot(aggn.reshape(NNODES_EN * b, FEATS), wn_ref[l],
                     preferred_element_type=f32) + bn_ref[l]
        hn = hn + jnp.tanh(zn).reshape(NNODES_EN, b, FEATS)

    ge = jnp.sum(he, axis=0)
    gn = jnp.sum(hn, axis=0)
    te = jnp.tanh(jnp.dot(ge, w1e_ref[...], preferred_element_type=f32) + b1e_ref[...])
    tn = jnp.tanh(jnp.dot(gn, w1n_ref[...], preferred_element_type=f32) + b1n_ref[...])
    ke = jnp.sum(te * w2e_ref[...], axis=1, keepdims=True) + b2e_ref[...]
    kn = jnp.sum(tn * w2n_ref[...], axis=1, keepdims=True) + b2n_ref[...]
    out_ref[...] = jnp.exp(ke + kn)


def _full(shape):
    nd = len(shape)
    return pl.BlockSpec(shape, lambda i, _n=nd: (0,) * _n)


@jax.jit
def kernel(pos, params):
    pe, pn = params['ee'], params['en']
    h0e = pe['node_emb'][jnp.asarray(EE_NTYPES)]
    embe = pe['edge_emb'][jnp.asarray(EE_ETYPE_PAIR)]
    h0n = pn['node_emb'][jnp.asarray(EN_NTYPES)]
    embn = pn['edge_emb'][jnp.asarray(EN_ETYPE_Q)]

    weights = [
        jnp.asarray(ATOM_POS),
        jnp.asarray(np.sqrt(GAMMA) * RBF_CENTERS).reshape(1, NRBF),
        jnp.asarray(EE_DIFF),
        h0e, embe, pe['w_rbf'], pe['w'], pe['b'],
        pe['w_out1'], pe['b_out1'].reshape(1, FEATS),
        pe['w_out2'].reshape(1, FEATS), pe['b_out2'].reshape(1, 1),
        h0n, embn, pn['w_rbf'], pn['w'], pn['b'],
        pn['w_out1'], pn['b_out1'].reshape(1, FEATS),
        pn['w_out2'].reshape(1, FEATS), pn['b_out2'].reshape(1, 1),
    ]

    nb = pos.shape[0]
    assert nb % BBLK == 0
    grid = (nb // BBLK,)

    return pl.pallas_call(
        _gnn_body,
        grid=grid,
        in_specs=[pl.BlockSpec((BBLK, NELEC * NDIM), lambda i: (i, 0))]
                 + [_full(w.shape) for w in weights],
        out_specs=pl.BlockSpec((BBLK, 1), lambda i: (i, 0)),
        out_shape=jax.ShapeDtypeStruct((nb, 1), jnp.float32),
        scratch_shapes=[pltpu.VMEM((NELEC * NELEC, BBLK, FEATS), jnp.float32)],
        compiler_params=pltpu.CompilerParams(
            dimension_semantics=("parallel",),
        ),
    )(pos, *weights)


# speedup vs baseline: 24.4455x; 1.0011x over previous
import numpy as np
from functools import partial

import jax
import jax.numpy as jnp
from jax.experimental import pallas as pl
from jax.experimental.pallas import tpu as pltpu

NUP = 8
NDOWN = 8
NELEC = NUP + NDOWN
NDIM = 3
ATOM_POS = np.array(
    [[0.0, 0.0, 0.0], [1.4, 0.0, 0.0], [0.0, 1.4, 0.0], [0.0, 0.0, 1.4]],
    dtype=np.float32,
)
NATOMS = ATOM_POS.shape[0]
NNODES_EN = NELEC + NATOMS
FEATS = 128
NRBF = 64
NLAYERS = 3
GAMMA = 10.0
RBF_CENTERS = np.linspace(0.0, 8.0, NRBF).astype(np.float32)

_spins = (np.arange(NELEC) >= NUP).astype(np.int32)
_rows, _cols = np.triu_indices(NELEC, k=1)
NPAIRS = len(_rows)
EE_ETYPE_PAIR = (_spins[_rows] + _spins[_cols]).astype(np.int32)
EE_DIFF = np.zeros((NDIM * NPAIRS, NELEC * NDIM), np.float32)
for _k in range(NDIM):
    EE_DIFF[_k * NPAIRS + np.arange(NPAIRS), NDIM * _rows + _k] = 1.0
    EE_DIFF[_k * NPAIRS + np.arange(NPAIRS), NDIM * _cols + _k] = -1.0
EE_NTYPES = _spins
_aq, _eq = np.divmod(np.arange(NATOMS * NELEC), NELEC)
EN_ETYPE_Q = (_spins[_eq] * NATOMS + _aq).astype(np.int32)
EN_NTYPES = np.concatenate([_spins, 2 + np.arange(NATOMS)]).astype(np.int32)

BBLK = 128


def _gnn_body(pos_ref, ap_ref, cen_ref, dee_ref,
              h0e_ref, embe_ref, wrbfe_ref, we_ref, be_ref,
              w1e_ref, b1e_ref, w2e_ref, b2e_ref,
              h0n_ref, embn_ref, wrbfn_ref, wn_ref, bn_ref,
              w1n_ref, b1n_ref, w2n_ref, b2n_ref,
              out_ref, e4_ref):
    b = pos_ref.shape[0]
    f32 = jnp.float32

    Xt = pos_ref[...].T
    X = Xt.reshape(NELEC, NDIM, b)
    ap = ap_ref[...]
    dif = jnp.dot(dee_ref[...], Xt,
                  preferred_element_type=f32).reshape(NDIM, NPAIRS, b)
    G = GAMMA
    d_ee = jnp.sqrt(G * jnp.sum(dif * dif, axis=0) + G * 1e-12)
    d2_en = jnp.zeros((NATOMS, NELEC, b), f32)
    for k in range(NDIM):
        Xk = X[:, k, :]
        dkn = Xk[None, :, :] - ap[:, k][:, None, None]
        d2_en = d2_en + dkn * dkn
    d_en = jnp.sqrt(G * d2_en + G * 1e-12).reshape(NATOMS * NELEC, b)

    cen = cen_ref[...][0]

    t_e = d_ee[:, :, None] - cen[None, None, :]
    rbf_e = jnp.exp(-(t_e * t_e))
    fe = jnp.dot(rbf_e.reshape(NPAIRS * b, NRBF), wrbfe_ref[...],
                 preferred_element_type=f32)
    fe = jnp.tanh(fe.reshape(NPAIRS, b, FEATS) + embe_ref[...][:, None, :])

    t_n = d_en[:, :, None] - cen[None, None, :]
    rbf_n = jnp.exp(-(t_n * t_n))
    fn = jnp.dot(rbf_n.reshape(NATOMS * NELEC * b, NRBF), wrbfn_ref[...],
                 preferred_element_type=f32)
    fn = jnp.tanh(fn.reshape(NATOMS * NELEC, b, FEATS) + embn_ref[...][:, None, :])
    EF = fn.reshape(NATOMS, NELEC, b, FEATS)

    zrow = jnp.zeros((b, FEATS), f32)
    for i in range(NELEC):
        e4_ref[i * NELEC + i] = zrow
    for p in range(NPAIRS):
        r = int(_rows[p]); c = int(_cols[p])
        row = fe[p]
        e4_ref[r * NELEC + c] = row
        e4_ref[c * NELEC + r] = row

    DB = 8
    he = jnp.broadcast_to(h0e_ref[...][:, None, :], (NELEC, b, FEATS))
    hn = jnp.broadcast_to(h0n_ref[...][:, None, :], (NNODES_EN, b, FEATS))
    for l in range(NLAYERS):
        if l == 0:
            hu = he[0]
            hv = he[NUP]
            agg_rows = []
            for d in range(NELEC):
                su = e4_ref[d * NELEC + 0]
                for s in range(1, NUP):
                    su = su + e4_ref[d * NELEC + s]
                sv = e4_ref[d * NELEC + NUP]
                for s in range(NUP + 1, NELEC):
                    sv = sv + e4_ref[d * NELEC + s]
                agg_rows.append(su * hu + sv * hv)
        else:
            agg_rows = []
            for d0 in range(0, NELEC, DB):
                accs = [None] * DB
                for s in range(NELEC):
                    hes = he[s]
                    for j in range(DB):
                        t = e4_ref[(d0 + j) * NELEC + s] * hes
                        accs[j] = t if accs[j] is None else accs[j] + t
                agg_rows += accs
        agg_e = jnp.stack(agg_rows)
        ze = jnp.dot(agg_e.reshape(NELEC * b, FEATS), we_ref[l],
                     preferred_element_type=f32) + be_ref[l]
        he = he + jnp.tanh(ze).reshape(NELEC, b, FEATS)

        hel = hn[:NELEC]
        hat = hn[NELEC:]
        el_rows = []
        for e0 in range(0, NELEC, DB):
            accs = [None] * DB
            for a in range(NATOMS):
                hata = hat[a]
                for j in range(DB):
                    t = EF[a, e0 + j] * hata
                    accs[j] = t if accs[j] is None else accs[j] + t
            el_rows += accs
        if l == 0:
            hnu = hel[0]
            hnv = hel[NUP]
            at_accs = []
            for a in range(NATOMS):
                su = EF[a, 0]
                for e in range(1, NUP):
                    su = su + EF[a, e]
                sv = EF[a, NUP]
                for e in range(NUP + 1, NELEC):
                    sv = sv + EF[a, e]
                at_accs.append(su * hnu + sv * hnv)
        else:
            at_accs = [None] * NATOMS
            for e in range(NELEC):
                hele = hel[e]
                for a in range(NATOMS):
                    t = EF[a, e] * hele
                    at_accs[a] = t if at_accs[a] is None else at_accs[a] + t
        aggn = jnp.stack(el_rows + at_accs)
        zn = jnp.dot(aggn.reshape(NNODES_EN * b, FEATS), wn_ref[l],
                     preferred_element_type=f32) + bn_ref[l]
        hn = hn + jnp.tanh(zn).reshape(NNODES_EN, b, FEATS)

    ge = jnp.sum(he, axis=0)
    gn = jnp.sum(hn, axis=0)
    te = jnp.tanh(jnp.dot(ge, w1e_ref[...], preferred_element_type=f32) + b1e_ref[...])
    tn = jnp.tanh(jnp.dot(gn, w1n_ref[...], preferred_element_type=f32) + b1n_ref[...])
    ke = jnp.sum(te * w2e_ref[...], axis=1, keepdims=True) + b2e_ref[...]
    kn = jnp.sum(tn * w2n_ref[...], axis=1, keepdims=True) + b2n_ref[...]
    out_ref[...] = jnp.exp(ke + kn)


def _full(shape):
    nd = len(shape)
    return pl.BlockSpec(shape, lambda i, _n=nd: (0,) * _n)


@jax.jit
def kernel(pos, params):
    pe, pn = params['ee'], params['en']
    h0e = pe['node_emb'][jnp.asarray(EE_NTYPES)]
    embe = pe['edge_emb'][jnp.asarray(EE_ETYPE_PAIR)]
    h0n = pn['node_emb'][jnp.asarray(EN_NTYPES)]
    embn = pn['edge_emb'][jnp.asarray(EN_ETYPE_Q)]

    weights = [
        jnp.asarray(ATOM_POS),
        jnp.asarray(np.sqrt(GAMMA) * RBF_CENTERS).reshape(1, NRBF),
        jnp.asarray(EE_DIFF),
        h0e, embe, pe['w_rbf'], pe['w'], pe['b'],
        pe['w_out1'], pe['b_out1'].reshape(1, FEATS),
        pe['w_out2'].reshape(1, FEATS), pe['b_out2'].reshape(1, 1),
        h0n, embn, pn['w_rbf'], pn['w'], pn['b'],
        pn['w_out1'], pn['b_out1'].reshape(1, FEATS),
        pn['w_out2'].reshape(1, FEATS), pn['b_out2'].reshape(1, 1),
    ]

    nb = pos.shape[0]
    assert nb % BBLK == 0
    grid = (nb // BBLK,)

    return pl.pallas_call(
        _gnn_body,
        grid=grid,
        in_specs=[pl.BlockSpec((BBLK, NELEC * NDIM), lambda i: (i, 0))]
                 + [_full(w.shape) for w in weights],
        out_specs=pl.BlockSpec((BBLK, 1), lambda i: (i, 0)),
        out_shape=jax.ShapeDtypeStruct((nb, 1), jnp.float32),
        scratch_shapes=[pltpu.VMEM((NELEC * NELEC, BBLK, FEATS), jnp.float32)],
        compiler_params=pltpu.CompilerParams(
            dimension_semantics=("parallel",),
        ),
    )(pos, *weights)
